# baseline (speedup 1.0000x reference)
import jax
import jax.numpy as jnp
from jax import lax
from jax.experimental import pallas as pl
from jax.experimental.pallas import tpu as pltpu
from jax.experimental.pallas import tpu_sc as plsc

_B, _N, _M = 8, 4096, 256
_NCH = 256
_NB = _N // _NCH
_BIGF = 1e30
_BIGI = 1 << 30

_NW = 32
_SLOTS = _B * 2 * _M
_SPW = _SLOTS // _NW
_LANES = 16


def _tc_body(shape_ref, skelT_ref, out_cd, out_i1, out_i2,
             cda, m1, i1, m2, i2):
    b = pl.program_id(0)
    nb = pl.program_id(1)

    blk = shape_ref[0]
    px, py, pz = blk[:, 0:1], blk[:, 1:2], blk[:, 2:3]
    sk = skelT_ref[0]
    sx, sy, sz = sk[0:1, :], sk[1:2, :], sk[2:3, :]

    dxx = px - sx
    dyy = py - sy
    dzz = pz - sz
    d2m = dxx * dxx + dyy * dyy + dzz * dzz

    c1 = jnp.min(d2m, axis=1, keepdims=True)
    cd_part = jnp.sum(jnp.sqrt(c1 + 1e-12), keepdims=True).reshape(1, 1)

    ri = lax.broadcasted_iota(jnp.int32, (_NCH, _M), 0) + (b * _N + nb * _NCH)
    bm1 = jnp.min(d2m, axis=0, keepdims=True)
    bi1 = jnp.min(jnp.where(d2m == bm1, ri, _BIGI), axis=0, keepdims=True)
    sel1 = ri == bi1
    mk = jnp.where(sel1, _BIGF, d2m)
    bm2 = jnp.min(mk, axis=0, keepdims=True)
    bi2 = jnp.min(jnp.where(mk == bm2, ri, _BIGI), axis=0, keepdims=True)

    @pl.when(nb == 0)
    def _init():
        m1[...] = jnp.full((1, _M), _BIGF)
        m2[...] = jnp.full((1, _M), _BIGF)
        i1[...] = jnp.zeros((1, _M), jnp.int32)
        i2[...] = jnp.zeros((1, _M), jnp.int32)

    @pl.when((b == 0) & (nb == 0))
    def _init_acc():
        cda[...] = jnp.zeros((1, 1), jnp.float32)

    rm1, ri1, rm2, ri2 = m1[...], i1[...], m2[...], i2[...]
    c1lt = bm1 < rm1
    nm1 = jnp.where(c1lt, bm1, rm1)
    nv1 = jnp.where(c1lt, bi1, ri1)
    cm = jnp.where(c1lt, rm1, rm2)
    cv = jnp.where(c1lt, ri1, ri2)
    cbm = jnp.where(c1lt, bm2, bm1)
    cbv = jnp.where(c1lt, bi2, bi1)
    c2lt = cbm < cm
    nm2 = jnp.where(c2lt, cbm, cm)
    nv2 = jnp.where(c2lt, cbv, cv)
    m1[...] = nm1
    i1[...] = nv1
    m2[...] = nm2
    i2[...] = nv2

    cda[...] = cda[...] + cd_part

    @pl.when(nb == _NB - 1)
    def _fin_batch():
        cd2v = jnp.sum(jnp.sqrt(m1[...] + 1e-12), keepdims=True).reshape(1, 1)
        cda[...] = cda[...] + cd2v
        out_i1[0] = i1[...]
        out_i2[0] = i2[...]

    @pl.when((b == _B - 1) & (nb == _NB - 1))
    def _emit():
        out_cd[...] = cda[...]


def _tc_call(shape_xyz, skelT):
    return pl.pallas_call(
        _tc_body,
        grid=(_B, _NB),
        in_specs=[
            pl.BlockSpec((1, _NCH, 6), lambda b, nb: (b, nb, 0)),
            pl.BlockSpec((1, 3, _M), lambda b, nb: (b, 0, 0)),
        ],
        out_specs=[
            pl.BlockSpec((1, 1), lambda b, nb: (0, 0)),
            pl.BlockSpec((1, 1, _M), lambda b, nb: (b, 0, 0)),
            pl.BlockSpec((1, 1, _M), lambda b, nb: (b, 0, 0)),
        ],
        out_shape=[
            jax.ShapeDtypeStruct((1, 1), jnp.float32),
            jax.ShapeDtypeStruct((_B, 1, _M), jnp.int32),
            jax.ShapeDtypeStruct((_B, 1, _M), jnp.int32),
        ],
        scratch_shapes=[
            pltpu.VMEM((1, 1), jnp.float32),
            pltpu.VMEM((1, _M), jnp.float32),
            pltpu.VMEM((1, _M), jnp.int32),
            pltpu.VMEM((1, _M), jnp.float32),
            pltpu.VMEM((1, _M), jnp.int32),
        ],
    )(shape_xyz, skelT)


def _sc_body(tbl_hbm, idx_hbm, nori_hbm, out_hbm,
             idx_v, row_idx_v, rows_v, nori_v, acc_v, sem):
    cid = lax.axis_index("c")
    sid = lax.axis_index("s")
    wid = cid * 16 + sid
    pltpu.sync_copy(idx_hbm.at[wid], idx_v)
    for j in range(_SPW // _LANES):
        sl = pl.ds(j * _LANES, _LANES)
        row_idx_v[sl] = lax.shift_right_logical(idx_v[sl], 3)
    gather = pltpu.async_copy(tbl_hbm.at[row_idx_v], rows_v, sem)
    pltpu.sync_copy(nori_hbm.at[wid], nori_v)
    gather.wait()
    acc = jnp.zeros((_LANES,), jnp.float32)
    ids0 = lax.broadcasted_iota(jnp.int32, (_LANES,), 0)
    for j in range(_SPW // _LANES):
        sl = pl.ds(j * _LANES, _LANES)
        ids = ids0 + (j * _LANES)
        sub = (idx_v[sl] & 7) * _LANES
        nx = plsc.load_gather(rows_v, [ids, sub])
        ny = plsc.load_gather(rows_v, [ids, sub + 1])
        nz = plsc.load_gather(rows_v, [ids, sub + 2])
        ox = nori_v[0, sl]
        oy = nori_v[1, sl]
        oz = nori_v[2, sl]
        acc = acc + jnp.abs(nx * ox + ny * oy + nz * oz)
    acc_v[...] = acc
    pltpu.sync_copy(acc_v, out_hbm.at[wid])


def _sc_call(tblpad, idx_w, nori_w):
    return pl.kernel(
        _sc_body,
        out_type=jax.ShapeDtypeStruct((_NW, _LANES), jnp.float32),
        mesh=plsc.VectorSubcoreMesh(core_axis_name="c", subcore_axis_name="s"),
        compiler_params=pltpu.CompilerParams(needs_layout_passes=False),
        scratch_types=[
            pltpu.VMEM((_SPW,), jnp.int32),
            pltpu.VMEM((_SPW,), jnp.int32),
            pltpu.VMEM((_SPW, 8 * _LANES), jnp.float32),
            pltpu.VMEM((3, _SPW), jnp.float32),
            pltpu.VMEM((_LANES,), jnp.float32),
            pltpu.SemaphoreType.DMA,
        ],
    )(tblpad, idx_w, nori_w)


def kernel(shape_xyz, skel_xyz, skel_nori):
    skelT = jnp.transpose(skel_xyz, (0, 2, 1))
    cd_raw, idx1, idx2 = _tc_call(shape_xyz, skelT)

    idx_w = jnp.concatenate([idx1, idx2], axis=1)
    idx_w = idx_w.reshape(_NW, _SPW)
    noriT = jnp.transpose(skel_nori, (0, 2, 1))
    nori_s = jnp.stack([noriT, noriT], axis=1)
    nori_w = jnp.transpose(nori_s, (2, 0, 1, 3)).reshape(3, _SLOTS)
    nori_w = nori_w.reshape(3, _NW, _SPW).transpose(1, 0, 2)
    tblpad = jnp.pad(shape_xyz[:, :, 3:6].reshape(_B * _N, 3),
                     ((0, 0), (0, _LANES - 3)))
    tblpad = tblpad.reshape(_B * _N // 8, 8 * _LANES)

    parts = _sc_call(tblpad, idx_w, nori_w)
    return cd_raw[0, 0] * 1e-4 + 0.001 * (jnp.sum(parts) / (2.0 * _B))


# speedup vs baseline: 1.1180x; 1.1180x over previous
import jax
import jax.numpy as jnp
from jax import lax
from jax.experimental import pallas as pl
from jax.experimental.pallas import tpu as pltpu
from jax.experimental.pallas import tpu_sc as plsc

_B, _N, _M = 8, 4096, 256
_NCH = 256
_NB = _N // _NCH
_BIGF = 1e30
_BIGI = 1 << 30

_NW = 32
_SLOTS = _B * 2 * _M
_SPW = _SLOTS // _NW
_LANES = 16


def _tc_body(shape_ref, skelT_ref, out_cd, out_idx,
             cda, m1, i1, m2, i2):
    b = pl.program_id(0)
    nb = pl.program_id(1)

    blk = shape_ref[0]
    px, py, pz = blk[:, 0:1], blk[:, 1:2], blk[:, 2:3]
    sk = skelT_ref[0]
    sx, sy, sz = sk[0:1, :], sk[1:2, :], sk[2:3, :]

    dxx = px - sx
    dyy = py - sy
    dzz = pz - sz
    d2m = dxx * dxx + dyy * dyy + dzz * dzz

    c1 = jnp.min(d2m, axis=1, keepdims=True)
    cd_part = jnp.sum(jnp.sqrt(c1 + 1e-12), keepdims=True).reshape(1, 1)

    ri = lax.broadcasted_iota(jnp.int32, (_NCH, _M), 0) + nb * _NCH
    bm1 = jnp.min(d2m, axis=0, keepdims=True)
    bi1 = jnp.min(jnp.where(d2m == bm1, ri, _BIGI), axis=0, keepdims=True)
    sel1 = ri == bi1
    mk = jnp.where(sel1, _BIGF, d2m)
    bm2 = jnp.min(mk, axis=0, keepdims=True)
    bi2 = jnp.min(jnp.where(mk == bm2, ri, _BIGI), axis=0, keepdims=True)

    @pl.when(nb == 0)
    def _init():
        m1[...] = jnp.full((1, _M), _BIGF)
        m2[...] = jnp.full((1, _M), _BIGF)
        i1[...] = jnp.zeros((1, _M), jnp.int32)
        i2[...] = jnp.zeros((1, _M), jnp.int32)

    @pl.when((b == 0) & (nb == 0))
    def _init_acc():
        cda[...] = jnp.zeros((1, 1), jnp.float32)

    rm1, ri1, rm2, ri2 = m1[...], i1[...], m2[...], i2[...]
    c1lt = bm1 < rm1
    nm1 = jnp.where(c1lt, bm1, rm1)
    nv1 = jnp.where(c1lt, bi1, ri1)
    cm = jnp.where(c1lt, rm1, rm2)
    cv = jnp.where(c1lt, ri1, ri2)
    cbm = jnp.where(c1lt, bm2, bm1)
    cbv = jnp.where(c1lt, bi2, bi1)
    c2lt = cbm < cm
    nm2 = jnp.where(c2lt, cbm, cm)
    nv2 = jnp.where(c2lt, cbv, cv)
    m1[...] = nm1
    i1[...] = nv1
    m2[...] = nm2
    i2[...] = nv2

    cda[...] = cda[...] + cd_part

    @pl.when(nb == _NB - 1)
    def _fin_batch():
        cd2v = jnp.sum(jnp.sqrt(m1[...] + 1e-12), keepdims=True).reshape(1, 1)
        cda[...] = cda[...] + cd2v
        out_idx[0, 0:1, :] = i1[...]
        out_idx[0, 1:2, :] = i2[...]

    @pl.when((b == _B - 1) & (nb == _NB - 1))
    def _emit():
        out_cd[...] = cda[...]


def _tc_call(shape_xyz, skelT):
    return pl.pallas_call(
        _tc_body,
        grid=(_B, _NB),
        in_specs=[
            pl.BlockSpec((1, _NCH, 6), lambda b, nb: (b, nb, 0)),
            pl.BlockSpec((1, 3, _M), lambda b, nb: (b, 0, 0)),
        ],
        out_specs=[
            pl.BlockSpec((1, 1), lambda b, nb: (0, 0)),
            pl.BlockSpec((1, 2, _M), lambda b, nb: (b, 0, 0)),
        ],
        out_shape=[
            jax.ShapeDtypeStruct((1, 1), jnp.float32),
            jax.ShapeDtypeStruct((_B, 2, _M), jnp.int32),
        ],
        scratch_shapes=[
            pltpu.VMEM((1, 1), jnp.float32),
            pltpu.VMEM((1, _M), jnp.float32),
            pltpu.VMEM((1, _M), jnp.int32),
            pltpu.VMEM((1, _M), jnp.float32),
            pltpu.VMEM((1, _M), jnp.int32),
        ],
    )(shape_xyz, skelT)


def _sc_body(shape_hbm, idx_hbm, noriT_hbm, out_hbm,
             pts_v, idx_v, nori_v, acc_v, sem):
    cid = lax.axis_index("c")
    sid = lax.axis_index("s")
    wid = cid * 16 + sid
    b = wid >> 2
    m0 = (wid & 1) * _SPW
    stage = pltpu.async_copy(
        shape_hbm.at[pl.ds(b * (_N * 6), _N * 6)], pts_v, sem)
    pltpu.sync_copy(idx_hbm.at[wid], idx_v)
    pltpu.sync_copy(noriT_hbm.at[b, :, pl.ds(m0, _SPW)], nori_v)
    stage.wait()
    acc = jnp.zeros((_LANES,), jnp.float32)
    for j in range(_SPW // _LANES):
        sl = pl.ds(j * _LANES, _LANES)
        r = idx_v[sl] * 6 + 3
        nx = plsc.load_gather(pts_v, [r])
        ny = plsc.load_gather(pts_v, [r + 1])
        nz = plsc.load_gather(pts_v, [r + 2])
        ox = nori_v[0, sl]
        oy = nori_v[1, sl]
        oz = nori_v[2, sl]
        acc = acc + jnp.abs(nx * ox + ny * oy + nz * oz)
    acc_v[...] = acc
    pltpu.sync_copy(acc_v, out_hbm.at[wid])


def _sc_call(shape_flat, idx_w, noriT):
    return pl.kernel(
        _sc_body,
        out_type=jax.ShapeDtypeStruct((_NW, _LANES), jnp.float32),
        mesh=plsc.VectorSubcoreMesh(core_axis_name="c", subcore_axis_name="s"),
        compiler_params=pltpu.CompilerParams(needs_layout_passes=False),
        scratch_types=[
            pltpu.VMEM((_N * 6,), jnp.float32),
            pltpu.VMEM((_SPW,), jnp.int32),
            pltpu.VMEM((3, _SPW), jnp.float32),
            pltpu.VMEM((_LANES,), jnp.float32),
            pltpu.SemaphoreType.DMA,
        ],
    )(shape_flat, idx_w, noriT)


def kernel(shape_xyz, skel_xyz, skel_nori):
    skelT = jnp.transpose(skel_xyz, (0, 2, 1))
    noriT = jnp.transpose(skel_nori, (0, 2, 1))
    cd_raw, idx = _tc_call(shape_xyz, skelT)

    idx_w = idx.reshape(_NW, _SPW)
    shape_flat = shape_xyz.reshape(_B * _N * 6)

    parts = _sc_call(shape_flat, idx_w, noriT)
    return cd_raw[0, 0] * 1e-4 + 0.001 * (jnp.sum(parts) / (2.0 * _B))


# speedup vs baseline: 1.3805x; 1.2348x over previous
import jax
import jax.numpy as jnp
from jax import lax
from jax.experimental import pallas as pl
from jax.experimental.pallas import tpu as pltpu
from jax.experimental.pallas import tpu_sc as plsc

_B, _N, _M = 8, 4096, 256
_NCH = 512
_NB = _N // _NCH
_KEYMASK = ~0xFFF
_IDXMASK = 0xFFF
_KEYMAX = 0x7FFFFFFF

_NW = 32
_SLOTS = _B * 2 * _M
_SPW = _SLOTS // _NW
_LANES = 16


def _tc_body(shape_ref, sknoT_ref, out_cd, out_idx, cda, k1, k2):
    b = pl.program_id(0)
    nb = pl.program_id(1)

    blk = shape_ref[0]
    p = blk[:, 0:3]
    sk = sknoT_ref[0, 0:3, :]

    cross = jnp.dot(p, sk, precision=lax.Precision.HIGHEST,
                    preferred_element_type=jnp.float32)
    p2 = jnp.sum(p * p, axis=1, keepdims=True)
    s2 = jnp.sum(sk * sk, axis=0, keepdims=True)
    d2m = jnp.maximum((p2 - 2.0 * cross) + s2, 0.0)

    c1 = jnp.min(d2m, axis=1, keepdims=True)
    cd_part = jnp.sum(jnp.sqrt(c1 + 1e-12), keepdims=True).reshape(1, 1)

    ri = lax.broadcasted_iota(jnp.int32, (_NCH, _M), 0) + nb * _NCH
    key = (lax.bitcast_convert_type(d2m, jnp.int32) & _KEYMASK) | ri
    bk1 = jnp.min(key, axis=0, keepdims=True)
    mk = jnp.where(key == bk1, _KEYMAX, key)
    bk2 = jnp.min(mk, axis=0, keepdims=True)

    @pl.when(nb == 0)
    def _init():
        k1[...] = jnp.full((1, _M), _KEYMAX, jnp.int32)
        k2[...] = jnp.full((1, _M), _KEYMAX, jnp.int32)

    @pl.when((b == 0) & (nb == 0))
    def _init_acc():
        cda[...] = jnp.zeros((1, 1), jnp.float32)

    rk1, rk2 = k1[...], k2[...]
    k1[...] = jnp.minimum(rk1, bk1)
    k2[...] = jnp.minimum(jnp.maximum(rk1, bk1), jnp.minimum(rk2, bk2))

    cda[...] = cda[...] + cd_part

    @pl.when(nb == _NB - 1)
    def _fin_batch():
        d2best = lax.bitcast_convert_type(k1[...] & _KEYMASK, jnp.float32)
        cd2v = jnp.sum(jnp.sqrt(d2best + 1e-12), keepdims=True).reshape(1, 1)
        cda[...] = cda[...] + cd2v
        out_idx[0, 0:1, :] = k1[...] & _IDXMASK
        out_idx[0, 1:2, :] = k2[...] & _IDXMASK

    @pl.when((b == _B - 1) & (nb == _NB - 1))
    def _emit():
        out_cd[...] = cda[...]


def _tc_call(shape_xyz, sknoT):
    return pl.pallas_call(
        _tc_body,
        grid=(_B, _NB),
        in_specs=[
            pl.BlockSpec((1, _NCH, 6), lambda b, nb: (b, nb, 0)),
            pl.BlockSpec((1, 6, _M), lambda b, nb: (b, 0, 0)),
        ],
        out_specs=[
            pl.BlockSpec((1, 1), lambda b, nb: (0, 0)),
            pl.BlockSpec((1, 2, _M), lambda b, nb: (b, 0, 0)),
        ],
        out_shape=[
            jax.ShapeDtypeStruct((1, 1), jnp.float32),
            jax.ShapeDtypeStruct((_B, 2, _M), jnp.int32),
        ],
        scratch_shapes=[
            pltpu.VMEM((1, 1), jnp.float32),
            pltpu.VMEM((1, _M), jnp.int32),
            pltpu.VMEM((1, _M), jnp.int32),
        ],
    )(shape_xyz, sknoT)


def _sc_body(shape_hbm, idx_hbm, sknoT_hbm, out_hbm,
             pts_v, idx_v, nori_v, acc_v, sem):
    cid = lax.axis_index("c")
    sid = lax.axis_index("s")
    wid = cid * 16 + sid
    b = wid >> 2
    m0 = (wid & 1) * _SPW
    stage = pltpu.async_copy(
        shape_hbm.at[pl.ds(b * (_N * 6), _N * 6)], pts_v, sem)
    pltpu.sync_copy(idx_hbm.at[wid], idx_v)
    pltpu.sync_copy(sknoT_hbm.at[b, pl.ds(3, 3), pl.ds(m0, _SPW)], nori_v)
    stage.wait()
    acc = jnp.zeros((_LANES,), jnp.float32)
    for j in range(_SPW // _LANES):
        sl = pl.ds(j * _LANES, _LANES)
        r = idx_v[sl] * 6 + 3
        nx = plsc.load_gather(pts_v, [r])
        ny = plsc.load_gather(pts_v, [r + 1])
        nz = plsc.load_gather(pts_v, [r + 2])
        ox = nori_v[0, sl]
        oy = nori_v[1, sl]
        oz = nori_v[2, sl]
        acc = acc + jnp.abs(nx * ox + ny * oy + nz * oz)
    acc_v[...] = acc
    pltpu.sync_copy(acc_v, out_hbm.at[wid])


def _sc_call(shape_flat, idx_w, sknoT):
    return pl.kernel(
        _sc_body,
        out_type=jax.ShapeDtypeStruct((_NW, _LANES), jnp.float32),
        mesh=plsc.VectorSubcoreMesh(core_axis_name="c", subcore_axis_name="s"),
        compiler_params=pltpu.CompilerParams(needs_layout_passes=False),
        scratch_types=[
            pltpu.VMEM((_N * 6,), jnp.float32),
            pltpu.VMEM((_SPW,), jnp.int32),
            pltpu.VMEM((3, _SPW), jnp.float32),
            pltpu.VMEM((_LANES,), jnp.float32),
            pltpu.SemaphoreType.DMA,
        ],
    )(shape_flat, idx_w, sknoT)


def kernel(shape_xyz, skel_xyz, skel_nori):
    skno = jnp.concatenate([skel_xyz, skel_nori], axis=-1)
    sknoT = jnp.transpose(skno, (0, 2, 1))
    cd_raw, idx = _tc_call(shape_xyz, sknoT)

    idx_w = idx.reshape(_NW, _SPW)
    shape_flat = shape_xyz.reshape(_B * _N * 6)

    parts = _sc_call(shape_flat, idx_w, sknoT)
    return cd_raw[0, 0] * 1e-4 + 0.001 * (jnp.sum(parts) / (2.0 * _B))


# speedup vs baseline: 1.4621x; 1.0591x over previous
import jax
import jax.numpy as jnp
from jax import lax
from jax.experimental import pallas as pl
from jax.experimental.pallas import tpu as pltpu
from jax.experimental.pallas import tpu_sc as plsc

_B, _N, _M = 8, 4096, 256
_NCH = 512
_NB = _N // _NCH
_KEYMASK = ~0xFFF
_IDXMASK = 0xFFF
_KEYMAX = 0x7FFFFFFF

_NW = 32
_SLOTS = _B * 2 * _M
_SPW = _SLOTS // _NW
_LANES = 16


def _tc_body(shape_ref, sknoT_ref, out_cd, out_idx, cda, k1, k2):
    b = pl.program_id(0)
    nb = pl.program_id(1)

    blk = shape_ref[0]
    px, py, pz = blk[:, 0:1], blk[:, 1:2], blk[:, 2:3]
    sk = sknoT_ref[0]
    sx, sy, sz = sk[0:1, :], sk[1:2, :], sk[2:3, :]

    dxx = px - sx
    dyy = py - sy
    dzz = pz - sz
    d2m = dxx * dxx + dyy * dyy + dzz * dzz

    c1 = jnp.min(d2m, axis=1, keepdims=True)
    cd_part = jnp.sum(jnp.sqrt(c1 + 1e-12), keepdims=True).reshape(1, 1)

    ri = lax.broadcasted_iota(jnp.int32, (_NCH, _M), 0) + nb * _NCH
    key = (lax.bitcast_convert_type(d2m, jnp.int32) & _KEYMASK) | ri
    bk1 = jnp.min(key, axis=0, keepdims=True)
    mk = jnp.where(key == bk1, _KEYMAX, key)
    bk2 = jnp.min(mk, axis=0, keepdims=True)

    @pl.when(nb == 0)
    def _init():
        k1[...] = jnp.full((1, _M), _KEYMAX, jnp.int32)
        k2[...] = jnp.full((1, _M), _KEYMAX, jnp.int32)

    @pl.when((b == 0) & (nb == 0))
    def _init_acc():
        cda[...] = jnp.zeros((1, 1), jnp.float32)

    rk1, rk2 = k1[...], k2[...]
    k1[...] = jnp.minimum(rk1, bk1)
    k2[...] = jnp.minimum(jnp.maximum(rk1, bk1), jnp.minimum(rk2, bk2))

    cda[...] = cda[...] + cd_part

    @pl.when(nb == _NB - 1)
    def _fin_batch():
        d2best = lax.bitcast_convert_type(k1[...] & _KEYMASK, jnp.float32)
        cd2v = jnp.sum(jnp.sqrt(d2best + 1e-12), keepdims=True).reshape(1, 1)
        cda[...] = cda[...] + cd2v
        out_idx[0, 0:1, :] = k1[...] & _IDXMASK
        out_idx[0, 1:2, :] = k2[...] & _IDXMASK

    @pl.when((b == _B - 1) & (nb == _NB - 1))
    def _emit():
        out_cd[...] = cda[...]


def _tc_call(shape_xyz, sknoT):
    return pl.pallas_call(
        _tc_body,
        grid=(_B, _NB),
        in_specs=[
            pl.BlockSpec((1, _NCH, 6), lambda b, nb: (b, nb, 0)),
            pl.BlockSpec((1, 6, _M), lambda b, nb: (b, 0, 0)),
        ],
        out_specs=[
            pl.BlockSpec((1, 1), lambda b, nb: (0, 0)),
            pl.BlockSpec((1, 2, _M), lambda b, nb: (b, 0, 0)),
        ],
        out_shape=[
            jax.ShapeDtypeStruct((1, 1), jnp.float32),
            jax.ShapeDtypeStruct((_B, 2, _M), jnp.int32),
        ],
        scratch_shapes=[
            pltpu.VMEM((1, 1), jnp.float32),
            pltpu.VMEM((1, _M), jnp.int32),
            pltpu.VMEM((1, _M), jnp.int32),
        ],
    )(shape_xyz, sknoT)


def _sc_body(shape_hbm, idx_hbm, sknoT_hbm, out_hbm,
             pts_v, idx_v, nori_v, acc_v, sem):
    cid = lax.axis_index("c")
    sid = lax.axis_index("s")
    wid = cid * 16 + sid
    b = wid >> 2
    m0 = (wid & 1) * _SPW
    stage = pltpu.async_copy(
        shape_hbm.at[pl.ds(b * (_N * 6), _N * 6)], pts_v, sem)
    pltpu.sync_copy(idx_hbm.at[wid], idx_v)
    pltpu.sync_copy(sknoT_hbm.at[b, pl.ds(3, 3), pl.ds(m0, _SPW)], nori_v)
    stage.wait()
    acc = jnp.zeros((_LANES,), jnp.float32)
    for j in range(_SPW // _LANES):
        sl = pl.ds(j * _LANES, _LANES)
        r = idx_v[sl] * 6 + 3
        nx = plsc.load_gather(pts_v, [r])
        ny = plsc.load_gather(pts_v, [r + 1])
        nz = plsc.load_gather(pts_v, [r + 2])
        ox = nori_v[0, sl]
        oy = nori_v[1, sl]
        oz = nori_v[2, sl]
        acc = acc + jnp.abs(nx * ox + ny * oy + nz * oz)
    acc_v[...] = acc
    pltpu.sync_copy(acc_v, out_hbm.at[wid])


def _sc_call(shape_flat, idx_w, sknoT):
    return pl.kernel(
        _sc_body,
        out_type=jax.ShapeDtypeStruct((_NW, _LANES), jnp.float32),
        mesh=plsc.VectorSubcoreMesh(core_axis_name="c", subcore_axis_name="s"),
        compiler_params=pltpu.CompilerParams(needs_layout_passes=False),
        scratch_types=[
            pltpu.VMEM((_N * 6,), jnp.float32),
            pltpu.VMEM((_SPW,), jnp.int32),
            pltpu.VMEM((3, _SPW), jnp.float32),
            pltpu.VMEM((_LANES,), jnp.float32),
            pltpu.SemaphoreType.DMA,
        ],
    )(shape_flat, idx_w, sknoT)


def kernel(shape_xyz, skel_xyz, skel_nori):
    skno = jnp.concatenate([skel_xyz, skel_nori], axis=-1)
    sknoT = jnp.transpose(skno, (0, 2, 1))
    cd_raw, idx = _tc_call(shape_xyz, sknoT)

    idx_w = idx.reshape(_NW, _SPW)
    shape_flat = shape_xyz.reshape(_B * _N * 6)

    parts = _sc_call(shape_flat, idx_w, sknoT)
    return cd_raw[0, 0] * 1e-4 + 0.001 * (jnp.sum(parts) / (2.0 * _B))
